# baseline (speedup 1.0000x reference)
import jax
import jax.numpy as jnp
from jax import lax
from jax.experimental import pallas as pl
from jax.experimental.pallas import tpu as pltpu
from jax.experimental.pallas import tpu_sc as plsc

XY = 100
STATES = XY * XY * XY
B = 16384

_INFO = plsc.get_sparse_core_info()
_NC, _NS, _L = _INFO.num_cores, _INFO.num_subcores, _INFO.num_lanes
_NW = _NC * _NS
_BPW = B // _NW
_CHUNK = 128
_NCHUNK = _BPW // _CHUNK
_GROUPS = _CHUNK // _L

_LN2 = 0.6931471805599453
_SQRT2 = 1.4142135623730951


def _log_f32(s):
    bits = lax.bitcast_convert_type(s, jnp.int32)
    e = (bits >> 23) - 127
    man = lax.bitcast_convert_type((bits & 0x007FFFFF) | 0x3F800000, jnp.float32)
    big = man > _SQRT2
    man = jnp.where(big, man * 0.5, man)
    ef = e.astype(jnp.float32) + jnp.where(big, 1.0, 0.0)
    z = (man - 1.0) / (man + 1.0)
    z2 = z * z
    p = 2.0 * z * (1.0 + z2 * (1.0 / 3.0 + z2 * (1.0 / 5.0 + z2 * (1.0 / 7.0))))
    return ef * _LN2 + p


def _body(table_ref, sn_ref, sp_ref, out_ref, sp_v, sn_v, idx_v, cols_v, out_v, sem):
    wid = lax.axis_index("s") * _NC + lax.axis_index("c")
    base = wid * _BPW

    for c in range(_NCHUNK):
        pltpu.sync_copy(sp_ref.at[pl.ds(base + c * _CHUNK, _CHUNK)], sp_v.at[c])
        pltpu.sync_copy(sn_ref.at[pl.ds(base + c * _CHUNK, _CHUNK)], sn_v.at[c])

    def idx_body(c, carry):
        for g in range(_GROUPS):
            sp7 = sp_v[c, pl.ds(g * _L, _L)] * 7
            for j in range(7):
                idx_v[c, j, pl.ds(g * _L, _L)] = sp7 + j
        return carry

    lax.fori_loop(0, _NCHUNK, idx_body, 0)

    copies = [
        pltpu.async_copy(table_ref.at[idx_v.at[c, j]], cols_v.at[c, j], sem)
        for c in range(_NCHUNK)
        for j in range(7)
    ]
    for cp in copies:
        cp.wait()

    def chunk_body(c, carry):
        for g in range(_GROUPS):
            sp = sp_v[c, pl.ds(g * _L, _L)]
            sn = sn_v[c, pl.ds(g * _L, _L)]
            zp = (sp.astype(jnp.float32) / float(XY * XY)).astype(jnp.int32)
            rp = sp - zp * (XY * XY)
            yp = (rp.astype(jnp.float32) / float(XY)).astype(jnp.int32)
            xp = rp - yp * XY
            zn = (sn.astype(jnp.float32) / float(XY * XY)).astype(jnp.int32)
            rn = sn - zn * (XY * XY)
            yn = (rn.astype(jnp.float32) / float(XY)).astype(jnp.int32)
            xn = rn - yn * XY
            dx = xn - xp
            dy = yn - yp
            dz = zn - zp

            x0 = dx == 0
            y0 = dy == 0
            z0 = dz == 0
            e = [
                x0 & y0 & z0,
                (dx == 1) & y0 & z0,
                (dx == -1) & y0 & z0,
                x0 & (dy == 1) & z0,
                x0 & (dy == -1) & z0,
                x0 & y0 & (dz == 1),
                x0 & y0 & (dz == 2),
            ]
            valid = e[0] | e[1] | e[2] | e[3] | e[4] | e[5] | e[6]

            cols = [cols_v[c, j, pl.ds(g * _L, _L)] for j in range(7)]
            m = cols[0]
            for j in range(1, 7):
                m = jnp.maximum(m, cols[j])
            s = jnp.exp(cols[0] - m)
            for j in range(1, 7):
                s = s + jnp.exp(cols[j] - m)
            chosen = cols[0]
            for j in range(1, 7):
                chosen = jnp.where(e[j], cols[j], chosen)
            res = chosen - m - _log_f32(s)
            res = jnp.where(valid, res, -jnp.inf)
            out_v[pl.ds(c * _CHUNK + g * _L, _L)] = res
        return carry

    lax.fori_loop(0, _NCHUNK, chunk_body, 0)
    pltpu.sync_copy(out_v, out_ref.at[pl.ds(base, _BPW)])


@jax.jit
def kernel(table, state_next, state_prev):
    mesh = plsc.VectorSubcoreMesh(core_axis_name="c", subcore_axis_name="s")
    f = pl.kernel(
        _body,
        out_type=jax.ShapeDtypeStruct((B,), jnp.float32),
        mesh=mesh,
        compiler_params=pltpu.CompilerParams(
            use_tc_tiling_on_sc=False, needs_layout_passes=False
        ),
        scratch_types=[
            pltpu.VMEM((_NCHUNK, _CHUNK), jnp.int32),
            pltpu.VMEM((_NCHUNK, _CHUNK), jnp.int32),
            pltpu.VMEM((_NCHUNK, 7, _CHUNK), jnp.int32),
            pltpu.VMEM((_NCHUNK, 7, _CHUNK), jnp.float32),
            pltpu.VMEM((_BPW,), jnp.float32),
            pltpu.SemaphoreType.DMA,
        ],
    )
    return f(jnp.reshape(table + jnp.float32(1e-30), (STATES * 7,)), state_next, state_prev)


# speedup vs baseline: 3.2599x; 3.2599x over previous
import jax
import jax.numpy as jnp
from jax import lax
from jax.experimental import pallas as pl
from jax.experimental.pallas import tpu as pltpu
from jax.experimental.pallas import tpu_sc as plsc

XY = 100
STATES = XY * XY * XY
B = 16384

_INFO = plsc.get_sparse_core_info()
_NC, _NS, _L = _INFO.num_cores, _INFO.num_subcores, _INFO.num_lanes
_NW = _NC * _NS
_BPW = B // _NW
_CHUNK = 128
_NCHUNK = _BPW // _CHUNK
_GROUPS = _CHUNK // _L

_LN2 = 0.6931471805599453
_SQRT2 = 1.4142135623730951


def _log_f32(s):
    bits = lax.bitcast_convert_type(s, jnp.int32)
    e = (bits >> 23) - 127
    man = lax.bitcast_convert_type((bits & 0x007FFFFF) | 0x3F800000, jnp.float32)
    big = man > _SQRT2
    man = jnp.where(big, man * 0.5, man)
    ef = e.astype(jnp.float32) + jnp.where(big, 1.0, 0.0)
    z = (man - 1.0) / (man + 1.0)
    z2 = z * z
    p = 2.0 * z * (1.0 + z2 * (1.0 / 3.0 + z2 * (1.0 / 5.0 + z2 * (1.0 / 7.0))))
    return ef * _LN2 + p


def _body(t0, t1, t2, t3, t4, t5, t6, sn_ref, sp_ref, out_ref, sp_v, sn_v, cols_v, out_v, sem):
    wid = lax.axis_index("s") * _NC + lax.axis_index("c")
    base = wid * _BPW

    for c in range(_NCHUNK):
        pltpu.sync_copy(sp_ref.at[pl.ds(base + c * _CHUNK, _CHUNK)], sp_v.at[c])
        pltpu.sync_copy(sn_ref.at[pl.ds(base + c * _CHUNK, _CHUNK)], sn_v.at[c])

    tcols = [t0, t1, t2, t3, t4, t5, t6]
    copies = [
        pltpu.async_copy(tcols[j].at[sp_v.at[c]], cols_v.at[c, j], sem)
        for c in range(_NCHUNK)
        for j in range(7)
    ]
    for cp in copies:
        cp.wait()

    def chunk_body(c, carry):
        for g in range(_GROUPS):
            sp = sp_v[c, pl.ds(g * _L, _L)]
            sn = sn_v[c, pl.ds(g * _L, _L)]
            zp = (sp.astype(jnp.float32) / float(XY * XY)).astype(jnp.int32)
            rp = sp - zp * (XY * XY)
            yp = (rp.astype(jnp.float32) / float(XY)).astype(jnp.int32)
            xp = rp - yp * XY
            zn = (sn.astype(jnp.float32) / float(XY * XY)).astype(jnp.int32)
            rn = sn - zn * (XY * XY)
            yn = (rn.astype(jnp.float32) / float(XY)).astype(jnp.int32)
            xn = rn - yn * XY
            dx = xn - xp
            dy = yn - yp
            dz = zn - zp

            x0 = dx == 0
            y0 = dy == 0
            z0 = dz == 0
            e = [
                x0 & y0 & z0,
                (dx == 1) & y0 & z0,
                (dx == -1) & y0 & z0,
                x0 & (dy == 1) & z0,
                x0 & (dy == -1) & z0,
                x0 & y0 & (dz == 1),
                x0 & y0 & (dz == 2),
            ]
            valid = e[0] | e[1] | e[2] | e[3] | e[4] | e[5] | e[6]

            cols = [cols_v[c, j, pl.ds(g * _L, _L)] for j in range(7)]
            m = cols[0]
            for j in range(1, 7):
                m = jnp.maximum(m, cols[j])
            s = jnp.exp(cols[0] - m)
            for j in range(1, 7):
                s = s + jnp.exp(cols[j] - m)
            chosen = cols[0]
            for j in range(1, 7):
                chosen = jnp.where(e[j], cols[j], chosen)
            res = chosen - m - _log_f32(s)
            res = jnp.where(valid, res, -jnp.inf)
            out_v[pl.ds(c * _CHUNK + g * _L, _L)] = res
        return carry

    lax.fori_loop(0, _NCHUNK, chunk_body, 0)
    pltpu.sync_copy(out_v, out_ref.at[pl.ds(base, _BPW)])


@jax.jit
def kernel(table, state_next, state_prev):
    mesh = plsc.VectorSubcoreMesh(core_axis_name="c", subcore_axis_name="s")
    f = pl.kernel(
        _body,
        out_type=jax.ShapeDtypeStruct((B,), jnp.float32),
        mesh=mesh,
        compiler_params=pltpu.CompilerParams(
            use_tc_tiling_on_sc=False, needs_layout_passes=False
        ),
        scratch_types=[
            pltpu.VMEM((_NCHUNK, _CHUNK), jnp.int32),
            pltpu.VMEM((_NCHUNK, _CHUNK), jnp.int32),
            pltpu.VMEM((_NCHUNK, 7, _CHUNK), jnp.float32),
            pltpu.VMEM((_BPW,), jnp.float32),
            pltpu.SemaphoreType.DMA,
        ],
    )
    return f(*(table[:, j] for j in range(7)), state_next, state_prev)
